# Initial kernel scaffold; baseline (speedup 1.0000x reference)
import jax
import jax.numpy as jnp
from jax.experimental import pallas as pl


def kernel(x):
    raise NotImplementedError("write your pallas kernel here")


# speedup vs baseline: 37.9585x; 37.9585x over previous
import jax
import jax.numpy as jnp
from jax.experimental import pallas as pl

_D = 2048
_K = 307


def _tc_body(x_ref, m_ref):
    xb = x_ref[...]
    ub = jax.lax.bitcast_convert_type(xb, jnp.int32) & jnp.int32(0x7FFFFFFF)

    def step(i, t):
        bit = jnp.int32(30) - i
        cand = t | (jnp.int32(1) << bit)
        cnt = jnp.sum((ub >= cand).astype(jnp.int32), axis=1, keepdims=True)
        return jnp.where(cnt >= _K, cand, t)

    t = jax.lax.fori_loop(0, 31, step, jnp.zeros((xb.shape[0], 1), jnp.int32))
    m_ref[...] = (ub >= t).astype(jnp.float32)


def kernel(x):
    B, T, D = x.shape
    R = B * T
    xf = x.reshape(R, D)
    BR = 512
    mask = pl.pallas_call(
        _tc_body,
        grid=(R // BR,),
        in_specs=[pl.BlockSpec((BR, D), lambda i: (i, 0))],
        out_specs=pl.BlockSpec((BR, D), lambda i: (i, 0)),
        out_shape=jax.ShapeDtypeStruct((R, D), jnp.float32),
    )(xf)
    return x, mask.reshape(B, T, D)
